# Initial kernel scaffold; baseline (speedup 1.0000x reference)
import jax
import jax.numpy as jnp
from jax.experimental import pallas as pl


def kernel(query, reference_points_l, reference_points_r, value, value_spatial_shapes, W_off, b_off, W_attn, b_attn, W_val, b_val, W_out, b_out):
    raise NotImplementedError("write your pallas kernel here")


# speedup vs baseline: 32.0130x; 32.0130x over previous
import functools

import jax
import jax.numpy as jnp
import numpy as np
from jax import lax
from jax.experimental import pallas as pl
from jax.experimental.pallas import tpu as pltpu
from jax.experimental.pallas import tpu_sc as plsc

_EMBED = 256
_NH = 8
_NL = 3
_NP = 4
_HD = _EMBED // _NH
_BS = 4
_NQ = 300
_NK = 2
_LQ = _NQ * _NK
_SHAPES = ((80, 80), (40, 40), (20, 20))
_SIZES = tuple(h * w for h, w in _SHAPES)
_OFFS = (0, 6400, 8000)
_LEN_V = 8400

_TAPS = _NL * 2 * _NP * 4
_NW = 32
_QCHUNK = 24
_NCHUNK = _LQ // _QCHUNK
_CT = _QCHUNK * _TAPS
_GSUB = 128
_NSUB = _CT // _GSUB


def _qproj_body(x_ref, wo_ref, bo_ref, wa_ref, ba_ref, blk_ref, so_ref, aw_ref):
    x = x_ref[...]
    so_ref[...] = (
        jnp.dot(x, wo_ref[...], preferred_element_type=jnp.float32) + bo_ref[...]
    )
    logits = jnp.dot(x, wa_ref[...], preferred_element_type=jnp.float32) + ba_ref[...]
    m = jnp.max(logits, axis=-1, keepdims=True)
    e = jnp.exp(logits - m)
    s = jnp.dot(e, blk_ref[...], preferred_element_type=jnp.float32)
    aw_ref[...] = e / s


def _matmul_bias_body(x_ref, w_ref, b_ref, o_ref):
    o_ref[...] = (
        jnp.dot(x_ref[...], w_ref[...], preferred_element_type=jnp.float32)
        + b_ref[...]
    )


def _tc_matmul_bias(x, w, b, mt):
    m, k = x.shape
    n = w.shape[1]
    grid = m // mt
    return pl.pallas_call(
        _matmul_bias_body,
        grid=(grid,),
        in_specs=[
            pl.BlockSpec((mt, k), lambda i: (i, 0)),
            pl.BlockSpec((k, n), lambda i: (0, 0)),
            pl.BlockSpec((1, n), lambda i: (0, 0)),
        ],
        out_specs=pl.BlockSpec((mt, n), lambda i: (i, 0)),
        out_shape=jax.ShapeDtypeStruct((m, n), jnp.float32),
    )(x, w, b.reshape(1, n))


def _tc_qproj(x, wo, bo, wa, ba):
    m, k = x.shape
    no, na = wo.shape[1], wa.shape[1]
    blk = (jnp.arange(na)[:, None] // (_NL * _NP)
           == jnp.arange(na)[None, :] // (_NL * _NP)).astype(jnp.float32)
    mt = 240
    return pl.pallas_call(
        _qproj_body,
        grid=(m // mt,),
        in_specs=[
            pl.BlockSpec((mt, k), lambda i: (i, 0)),
            pl.BlockSpec((k, no), lambda i: (0, 0)),
            pl.BlockSpec((1, no), lambda i: (0, 0)),
            pl.BlockSpec((k, na), lambda i: (0, 0)),
            pl.BlockSpec((1, na), lambda i: (0, 0)),
            pl.BlockSpec((na, na), lambda i: (0, 0)),
        ],
        out_specs=[
            pl.BlockSpec((mt, no), lambda i: (i, 0)),
            pl.BlockSpec((mt, na), lambda i: (i, 0)),
        ],
        out_shape=[
            jax.ShapeDtypeStruct((m, no), jnp.float32),
            jax.ShapeDtypeStruct((m, na), jnp.float32),
        ],
    )(x, wo, bo.reshape(1, no), wa, ba.reshape(1, na), blk)


def _sc_sample_body(table, idxs, wts, out, idx_v, wts_v, rows_v, out_v, sem):
    w = lax.axis_index("s") * 2 + lax.axis_index("c")

    def chunk_body(g, carry):
        pltpu.sync_copy(idxs.at[w, g], idx_v)
        pltpu.sync_copy(wts.at[w, pl.ds(g * _CT, _CT)], wts_v)
        copies = [
            pltpu.async_copy(
                table.at[idx_v.at[j]], rows_v.at[pl.ds(j * _GSUB, _GSUB)], sem
            )
            for j in range(_NSUB)
        ]
        for cp in copies:
            cp.wait()

        def q_body(q, c2):
            def g_body(g, acc):
                a0, a1 = acc
                base = q * _TAPS + g * 16
                wv16 = wts_v[pl.ds(base, 16)]
                for k in range(16):
                    wk = wv16[k]
                    a0 = a0 + wk * rows_v[base + k, pl.ds(0, 16)]
                    a1 = a1 + wk * rows_v[base + k, pl.ds(16, 16)]
                return (a0, a1)

            z = jnp.zeros((16,), jnp.float32)
            a0, a1 = lax.fori_loop(0, _TAPS // 16, g_body, (z, z))
            out_v[q, pl.ds(0, 16)] = a0
            out_v[q, pl.ds(16, 16)] = a1
            return c2

        lax.fori_loop(0, _QCHUNK, q_body, 0)
        pltpu.sync_copy(out_v, out.at[w, pl.ds(g * _QCHUNK, _QCHUNK)])
        return carry

    lax.fori_loop(0, _NCHUNK, chunk_body, 0)


@functools.lru_cache(maxsize=None)
def _sc_sample_fn():
    return pl.kernel(
        _sc_sample_body,
        out_type=jax.ShapeDtypeStruct((_NW, _LQ, _HD), jnp.float32),
        mesh=plsc.VectorSubcoreMesh(core_axis_name="c", subcore_axis_name="s"),
        scratch_types=[
            pltpu.VMEM((_NSUB, _GSUB), jnp.int32),
            pltpu.VMEM((_CT,), jnp.float32),
            pltpu.VMEM((_CT, _HD), jnp.float32),
            pltpu.VMEM((_QCHUNK, _HD), jnp.float32),
            pltpu.SemaphoreType.DMA,
        ],
        compiler_params=pltpu.CompilerParams(use_tc_tiling_on_sc=False),
    )


def _sc_sample(table, idxs, wts):
    return _sc_sample_fn()(table, idxs, wts)


def _build_taps(so, aw, rp_l, rp_r):
    bs, lq = _BS, _LQ
    rpl = jnp.transpose(rp_l, (0, 1, 3, 2, 4)).reshape(bs, lq, _NL, 2)
    rpr = jnp.transpose(rp_r, (0, 1, 3, 2, 4)).reshape(bs, lq, _NL, 2)
    wdim = jnp.asarray([float(w) for _, w in _SHAPES], jnp.float32)
    hdim = jnp.asarray([float(h) for h, _ in _SHAPES], jnp.float32)
    norm = jnp.stack([wdim, hdim], -1)

    locs_l = rpl[:, :, None, :, None, :] + so[..., :2] / norm[None, None, None, :, None, :]
    locs_r = rpr[:, :, None, :, None, :] + so[..., 2:] / norm[None, None, None, :, None, :]
    locs = jnp.concatenate([locs_l, locs_r], axis=-2)

    ix = locs[..., 0] * wdim[None, None, None, :, None] - 0.5
    iy = locs[..., 1] * hdim[None, None, None, :, None] - 0.5
    x0f = jnp.floor(ix)
    y0f = jnp.floor(iy)
    wx = ix - x0f
    wy = iy - y0f
    x0 = x0f.astype(jnp.int32)
    y0 = y0f.astype(jnp.int32)

    aw2 = jnp.concatenate([aw, aw], axis=-1)
    wi = jnp.asarray([w for _, w in _SHAPES], jnp.int32)[None, None, None, :, None]
    hi = jnp.asarray([h for h, _ in _SHAPES], jnp.int32)[None, None, None, :, None]
    offs = jnp.asarray(_OFFS, jnp.int32)[None, None, None, :, None]
    boff = (jnp.arange(bs, dtype=jnp.int32) * _LEN_V)[:, None, None, None, None]
    hoff = jnp.arange(_NH, dtype=jnp.int32)[None, None, :, None, None]

    idx_taps, wt_taps = [], []
    for dy in (0, 1):
        for dx in (0, 1):
            xt = x0 + dx
            yt = y0 + dy
            valid = ((xt >= 0) & (xt < wi) & (yt >= 0) & (yt < hi))
            wtap = ((wx if dx else (1.0 - wx)) * (wy if dy else (1.0 - wy))
                    * valid.astype(jnp.float32) * aw2)
            pix = (boff + offs
                   + jnp.clip(yt, 0, hi - 1) * wi
                   + jnp.clip(xt, 0, wi - 1))
            idx_taps.append(pix * _NH + hoff)
            wt_taps.append(wtap)

    idx = jnp.stack(idx_taps, axis=-1)
    wts = jnp.stack(wt_taps, axis=-1)
    idx = jnp.transpose(idx, (0, 2, 1, 3, 4, 5)).reshape(_NW, _NCHUNK, _NSUB, _GSUB)
    wts = jnp.transpose(wts, (0, 2, 1, 3, 4, 5)).reshape(_NW, lq * _TAPS)
    return idx, wts, locs_l, locs_r


def kernel(query, reference_points_l, reference_points_r, value,
           value_spatial_shapes, W_off, b_off, W_attn, b_attn, W_val, b_val,
           W_out, b_out):
    del value_spatial_shapes
    bs, lq, _ = query.shape

    q2d = query.reshape(bs * lq, _EMBED)
    so2d, aw2d = _tc_qproj(q2d, W_off, b_off, W_attn, b_attn)
    so = so2d.reshape(bs, lq, _NH, _NL, _NP, 4)
    aw = aw2d.reshape(bs, lq, _NH, _NL, _NP)

    v2d = _tc_matmul_bias(value.reshape(bs * _LEN_V, 2 * _EMBED), W_val, b_val, 480)
    table = v2d.reshape(bs * _LEN_V * _NH, _HD)

    idx, wts, locs_l, locs_r = _build_taps(so, aw, reference_points_l,
                                           reference_points_r)

    sampled = _sc_sample(table, idx, wts)
    out2d = jnp.transpose(sampled.reshape(_BS, _NH, lq, _HD), (0, 2, 1, 3))
    out2d = out2d.reshape(bs * lq, _EMBED)
    out = _tc_matmul_bias(out2d, W_out, b_out, 240).reshape(bs, lq, _EMBED)

    tp = _NH * _NL * _NP
    sl_l = locs_l.reshape(bs, _NQ, _NK, tp, 2)[:, :, 0]
    sl_r = locs_r.reshape(bs, _NQ, _NK, tp, 2)[:, :, 0]
    aw_log = aw.reshape(bs, _NQ, _NK, tp)[:, :, 0]
    log_info = (lax.stop_gradient(sl_l), lax.stop_gradient(aw_log),
                lax.stop_gradient(sl_r), lax.stop_gradient(aw_log))
    return out, log_info


# speedup vs baseline: 43.4648x; 1.3577x over previous
import functools

import jax
import jax.numpy as jnp
import numpy as np
from jax import lax
from jax.experimental import pallas as pl
from jax.experimental.pallas import tpu as pltpu
from jax.experimental.pallas import tpu_sc as plsc

_EMBED = 256
_NH = 8
_NL = 3
_NP = 4
_HD = _EMBED // _NH
_BS = 4
_NQ = 300
_NK = 2
_LQ = _NQ * _NK
_SHAPES = ((80, 80), (40, 40), (20, 20))
_SIZES = tuple(h * w for h, w in _SHAPES)
_OFFS = (0, 6400, 8000)
_LEN_V = 8400

_TAPS = _NL * 2 * _NP * 4
_NW = 32
_QCHUNK = 12
_NCHUNK = _LQ // _QCHUNK
_CT = _QCHUNK * _TAPS
_GSUB = 128
_NSUB = _CT // _GSUB


def _qproj_body(x_ref, wo_ref, bo_ref, wa_ref, ba_ref, blk_ref, so_ref, aw_ref):
    x = x_ref[...]
    so_ref[...] = (
        jnp.dot(x, wo_ref[...], preferred_element_type=jnp.float32) + bo_ref[...]
    )
    logits = jnp.dot(x, wa_ref[...], preferred_element_type=jnp.float32) + ba_ref[...]
    m = jnp.max(logits, axis=-1, keepdims=True)
    e = jnp.exp(logits - m)
    s = jnp.dot(e, blk_ref[...], preferred_element_type=jnp.float32)
    aw_ref[...] = e / s


def _matmul_bias_body(x_ref, w_ref, b_ref, o_ref):
    o_ref[...] = (
        jnp.dot(x_ref[...], w_ref[...], preferred_element_type=jnp.float32)
        + b_ref[...]
    )


def _tc_matmul_bias(x, w, b, mt):
    m, k = x.shape
    n = w.shape[1]
    grid = m // mt
    return pl.pallas_call(
        _matmul_bias_body,
        grid=(grid,),
        in_specs=[
            pl.BlockSpec((mt, k), lambda i: (i, 0)),
            pl.BlockSpec((k, n), lambda i: (0, 0)),
            pl.BlockSpec((1, n), lambda i: (0, 0)),
        ],
        out_specs=pl.BlockSpec((mt, n), lambda i: (i, 0)),
        out_shape=jax.ShapeDtypeStruct((m, n), jnp.float32),
    )(x, w, b.reshape(1, n))


def _tc_qproj(x, wo, bo, wa, ba):
    m, k = x.shape
    no, na = wo.shape[1], wa.shape[1]
    blk = (jnp.arange(na)[:, None] // (_NL * _NP)
           == jnp.arange(na)[None, :] // (_NL * _NP)).astype(jnp.float32)
    mt = 240
    return pl.pallas_call(
        _qproj_body,
        grid=(m // mt,),
        in_specs=[
            pl.BlockSpec((mt, k), lambda i: (i, 0)),
            pl.BlockSpec((k, no), lambda i: (0, 0)),
            pl.BlockSpec((1, no), lambda i: (0, 0)),
            pl.BlockSpec((k, na), lambda i: (0, 0)),
            pl.BlockSpec((1, na), lambda i: (0, 0)),
            pl.BlockSpec((na, na), lambda i: (0, 0)),
        ],
        out_specs=[
            pl.BlockSpec((mt, no), lambda i: (i, 0)),
            pl.BlockSpec((mt, na), lambda i: (i, 0)),
        ],
        out_shape=[
            jax.ShapeDtypeStruct((m, no), jnp.float32),
            jax.ShapeDtypeStruct((m, na), jnp.float32),
        ],
    )(x, wo, bo.reshape(1, no), wa, ba.reshape(1, na), blk)


def _sc_sample_body(table, idxs, wts, out, idx_v, wts_v, rows_v, out_v, sems):
    w = lax.axis_index("s") * 2 + lax.axis_index("c")
    b = w // _NH
    h = w % _NH
    sem0, sem1 = sems
    psem = (sem0, sem1)

    def issue(g, p):
        pltpu.sync_copy(idxs.at[w, g], idx_v.at[p])
        for j in range(_NSUB):
            pltpu.async_copy(
                table.at[idx_v.at[p, j]],
                rows_v.at[p, pl.ds(j * _GSUB, _GSUB)],
                psem[p],
            )
        pltpu.async_copy(wts.at[w, pl.ds(g * _CT, _CT)], wts_v.at[p], psem[p])

    def drain(p):
        for j in range(_NSUB):
            pltpu.make_async_copy(
                table.at[idx_v.at[p, j]],
                rows_v.at[p, pl.ds(j * _GSUB, _GSUB)],
                psem[p],
            ).wait()
        pltpu.make_async_copy(
            wts.at[w, pl.ds(0, _CT)], wts_v.at[p], psem[p]
        ).wait()

    def compute(g, p):
        def q_body(q, c2):
            def g_body(gg, acc):
                a0, a1 = acc
                base = q * _TAPS + gg * 16
                wv16 = wts_v[p, pl.ds(base, 16)]
                for k in range(16):
                    wk = wv16[k]
                    a0 = a0 + wk * rows_v[p, base + k, pl.ds(0, 16)]
                    a1 = a1 + wk * rows_v[p, base + k, pl.ds(16, 16)]
                return (a0, a1)

            z = jnp.zeros((16,), jnp.float32)
            a0, a1 = lax.fori_loop(0, _TAPS // 16, g_body, (z, z))
            out_v[q, pl.ds(0, 16)] = a0
            out_v[q, pl.ds(16, 16)] = a1
            return c2

        lax.fori_loop(0, _QCHUNK, q_body, 0)
        pltpu.sync_copy(out_v, out.at[b, pl.ds(g * _QCHUNK, _QCHUNK), h])

    issue(0, 0)

    def pair_body(o, carry):
        g0 = 2 * o
        issue(g0 + 1, 1)
        drain(0)
        compute(g0, 0)

        @pl.when(o < _NCHUNK // 2 - 1)
        def _():
            issue(g0 + 2, 0)

        drain(1)
        compute(g0 + 1, 1)
        return carry

    lax.fori_loop(0, _NCHUNK // 2, pair_body, 0)


@functools.lru_cache(maxsize=None)
def _sc_sample_fn():
    return pl.kernel(
        _sc_sample_body,
        out_type=jax.ShapeDtypeStruct((_BS, _LQ, _NH, _HD), jnp.float32),
        mesh=plsc.VectorSubcoreMesh(core_axis_name="c", subcore_axis_name="s"),
        scratch_types=[
            pltpu.VMEM((2, _NSUB, _GSUB), jnp.int32),
            pltpu.VMEM((2, _CT), jnp.float32),
            pltpu.VMEM((2, _CT, _HD), jnp.float32),
            pltpu.VMEM((_QCHUNK, _HD), jnp.float32),
            (pltpu.SemaphoreType.DMA, pltpu.SemaphoreType.DMA),
        ],
        compiler_params=pltpu.CompilerParams(use_tc_tiling_on_sc=False),
    )


def _sc_sample(table, idxs, wts):
    return _sc_sample_fn()(table, idxs, wts)


def _build_taps(so, aw, rp_l, rp_r):
    bs, lq = _BS, _LQ
    rpl = jnp.transpose(rp_l, (0, 1, 3, 2, 4)).reshape(bs, lq, _NL, 2)
    rpr = jnp.transpose(rp_r, (0, 1, 3, 2, 4)).reshape(bs, lq, _NL, 2)
    wdim = jnp.asarray([float(w) for _, w in _SHAPES], jnp.float32)
    hdim = jnp.asarray([float(h) for h, _ in _SHAPES], jnp.float32)
    norm = jnp.stack([wdim, hdim], -1)

    rpl_b = rpl[:, None, :, :, None, :]
    rpr_b = rpr[:, None, :, :, None, :]
    nrm = norm[None, None, None, :, None, :]
    locs_l = rpl_b + so[..., :2] / nrm
    locs_r = rpr_b + so[..., 2:] / nrm
    locs = jnp.concatenate([locs_l, locs_r], axis=-2)

    ix = locs[..., 0] * wdim[None, None, None, :, None] - 0.5
    iy = locs[..., 1] * hdim[None, None, None, :, None] - 0.5
    x0f = jnp.floor(ix)
    y0f = jnp.floor(iy)
    wx = ix - x0f
    wy = iy - y0f
    x0 = x0f.astype(jnp.int32)
    y0 = y0f.astype(jnp.int32)

    aw2 = jnp.concatenate([aw, aw], axis=-1)
    wi = jnp.asarray([w for _, w in _SHAPES], jnp.int32)[None, None, None, :, None]
    hi = jnp.asarray([h for h, _ in _SHAPES], jnp.int32)[None, None, None, :, None]
    offs = jnp.asarray(_OFFS, jnp.int32)[None, None, None, :, None]
    boff = (jnp.arange(bs, dtype=jnp.int32) * _LEN_V)[:, None, None, None, None]
    hoff = jnp.arange(_NH, dtype=jnp.int32)[None, :, None, None, None]

    idx_taps, wt_taps = [], []
    for dy in (0, 1):
        for dx in (0, 1):
            xt = x0 + dx
            yt = y0 + dy
            valid = ((xt >= 0) & (xt < wi) & (yt >= 0) & (yt < hi))
            wtap = ((wx if dx else (1.0 - wx)) * (wy if dy else (1.0 - wy))
                    * valid.astype(jnp.float32) * aw2)
            pix = (boff + offs
                   + jnp.clip(yt, 0, hi - 1) * wi
                   + jnp.clip(xt, 0, wi - 1))
            idx_taps.append(pix * _NH + hoff)
            wt_taps.append(wtap)

    idx = jnp.stack(idx_taps, axis=-1)
    wts = jnp.stack(wt_taps, axis=-1)
    idx = idx.reshape(_NW, _NCHUNK, _NSUB, _GSUB)
    wts = wts.reshape(_NW, lq * _TAPS)
    return idx, wts, locs_l, locs_r


def kernel(query, reference_points_l, reference_points_r, value,
           value_spatial_shapes, W_off, b_off, W_attn, b_attn, W_val, b_val,
           W_out, b_out):
    del value_spatial_shapes
    bs, lq, _ = query.shape

    q2d = query.reshape(bs * lq, _EMBED)
    so2d, aw2d = _tc_qproj(q2d, W_off, b_off, W_attn, b_attn)
    so = jnp.transpose(so2d.reshape(bs, lq, _NH, _NL * _NP * 4), (0, 2, 1, 3))
    so = so.reshape(bs, _NH, lq, _NL, _NP, 4)
    aw = jnp.transpose(aw2d.reshape(bs, lq, _NH, _NL * _NP), (0, 2, 1, 3))
    aw = aw.reshape(bs, _NH, lq, _NL, _NP)

    v2d = _tc_matmul_bias(value.reshape(bs * _LEN_V, 2 * _EMBED), W_val, b_val, 480)
    table = v2d.reshape(bs * _LEN_V * _NH, _HD)

    idx, wts, locs_l, locs_r = _build_taps(so, aw, reference_points_l,
                                           reference_points_r)

    sampled = _sc_sample(table, idx, wts)
    out2d = sampled.reshape(bs * lq, _EMBED)
    out = _tc_matmul_bias(out2d, W_out, b_out, 240).reshape(bs, lq, _EMBED)

    tp = _NH * _NL * _NP
    sl_l = jnp.transpose(locs_l[:, :, ::_NK], (0, 2, 1, 3, 4, 5)).reshape(
        bs, _NQ, tp, 2)
    sl_r = jnp.transpose(locs_r[:, :, ::_NK], (0, 2, 1, 3, 4, 5)).reshape(
        bs, _NQ, tp, 2)
    aw_log = jnp.transpose(aw[:, :, ::_NK], (0, 2, 1, 3, 4)).reshape(
        bs, _NQ, tp)
    log_info = (lax.stop_gradient(sl_l), lax.stop_gradient(aw_log),
                lax.stop_gradient(sl_r), lax.stop_gradient(aw_log))
    return out, log_info


# speedup vs baseline: 43.4664x; 1.0000x over previous
import functools

import jax
import jax.numpy as jnp
import numpy as np
from jax import lax
from jax.experimental import pallas as pl
from jax.experimental.pallas import tpu as pltpu
from jax.experimental.pallas import tpu_sc as plsc

_EMBED = 256
_NH = 8
_NL = 3
_NP = 4
_HD = _EMBED // _NH
_BS = 4
_NQ = 300
_NK = 2
_LQ = _NQ * _NK
_SHAPES = ((80, 80), (40, 40), (20, 20))
_SIZES = tuple(h * w for h, w in _SHAPES)
_OFFS = (0, 6400, 8000)
_LEN_V = 8400

_TAPS = _NL * 2 * _NP * 4
_NW = 32
_QCHUNK = 12
_NCHUNK = _LQ // _QCHUNK
_CT = _QCHUNK * _TAPS
_GSUB = 128
_NSUB = _CT // _GSUB


def _qproj_body(x_ref, wo_ref, bo_ref, wa_ref, ba_ref, blk_ref, so_ref, aw_ref):
    x = x_ref[...]
    so_ref[...] = (
        jnp.dot(x, wo_ref[...], preferred_element_type=jnp.float32) + bo_ref[...]
    )
    logits = jnp.dot(x, wa_ref[...], preferred_element_type=jnp.float32) + ba_ref[...]
    m = jnp.max(logits, axis=-1, keepdims=True)
    e = jnp.exp(logits - m)
    s = jnp.dot(e, blk_ref[...], preferred_element_type=jnp.float32)
    aw_ref[...] = e / s


def _matmul_bias_body(x_ref, w_ref, b_ref, o_ref):
    acc = (
        jnp.dot(x_ref[...], w_ref[...], preferred_element_type=jnp.float32)
        + b_ref[...]
    )
    o_ref[...] = acc.astype(o_ref.dtype)


def _tc_matmul_bias(x, w, b, mt, out_dtype=jnp.float32):
    m, k = x.shape
    n = w.shape[1]
    grid = m // mt
    return pl.pallas_call(
        _matmul_bias_body,
        grid=(grid,),
        in_specs=[
            pl.BlockSpec((mt, k), lambda i: (i, 0)),
            pl.BlockSpec((k, n), lambda i: (0, 0)),
            pl.BlockSpec((1, n), lambda i: (0, 0)),
        ],
        out_specs=pl.BlockSpec((mt, n), lambda i: (i, 0)),
        out_shape=jax.ShapeDtypeStruct((m, n), out_dtype),
    )(x, w, b.reshape(1, n))


def _tc_qproj(x, wo, bo, wa, ba):
    m, k = x.shape
    no, na = wo.shape[1], wa.shape[1]
    blk = (jnp.arange(na)[:, None] // (_NL * _NP)
           == jnp.arange(na)[None, :] // (_NL * _NP)).astype(jnp.float32)
    mt = 240
    return pl.pallas_call(
        _qproj_body,
        grid=(m // mt,),
        in_specs=[
            pl.BlockSpec((mt, k), lambda i: (i, 0)),
            pl.BlockSpec((k, no), lambda i: (0, 0)),
            pl.BlockSpec((1, no), lambda i: (0, 0)),
            pl.BlockSpec((k, na), lambda i: (0, 0)),
            pl.BlockSpec((1, na), lambda i: (0, 0)),
            pl.BlockSpec((na, na), lambda i: (0, 0)),
        ],
        out_specs=[
            pl.BlockSpec((mt, no), lambda i: (i, 0)),
            pl.BlockSpec((mt, na), lambda i: (i, 0)),
        ],
        out_shape=[
            jax.ShapeDtypeStruct((m, no), jnp.float32),
            jax.ShapeDtypeStruct((m, na), jnp.float32),
        ],
    )(x, wo, bo.reshape(1, no), wa, ba.reshape(1, na), blk)


def _sc_sample_body(table, idxs, wts, out, idx_v, wts_v, rows_v, out_v, sems):
    w = lax.axis_index("s") * 2 + lax.axis_index("c")
    b = w // _NH
    h = w % _NH
    sem0, sem1 = sems
    psem = (sem0, sem1)

    def issue(g, p):
        pltpu.sync_copy(idxs.at[w, g], idx_v.at[p])
        for j in range(_NSUB):
            pltpu.async_copy(
                table.at[idx_v.at[p, j]],
                rows_v.at[p, pl.ds(j * _GSUB, _GSUB)],
                psem[p],
            )
        pltpu.async_copy(wts.at[w, pl.ds(g * _CT, _CT)], wts_v.at[p], psem[p])

    def drain(p):
        for j in range(_NSUB):
            pltpu.make_async_copy(
                table.at[idx_v.at[p, j]],
                rows_v.at[p, pl.ds(j * _GSUB, _GSUB)],
                psem[p],
            ).wait()
        pltpu.make_async_copy(
            wts.at[w, pl.ds(0, _CT)], wts_v.at[p], psem[p]
        ).wait()

    def compute(g, p):
        def q_body(q, c2):
            def g_body(gg, acc):
                a0, a1 = acc
                base = q * _TAPS + gg * 16
                wv16 = wts_v[p, pl.ds(base, 16)]
                for k in range(16):
                    wk = wv16[k]
                    a0 = a0 + wk * rows_v[p, base + k, pl.ds(0, 16)]
                    a1 = a1 + wk * rows_v[p, base + k, pl.ds(16, 16)]
                return (a0, a1)

            z = jnp.zeros((16,), jnp.float32)
            a0, a1 = lax.fori_loop(0, _TAPS // 16, g_body, (z, z))
            out_v[q, pl.ds(0, 16)] = a0
            out_v[q, pl.ds(16, 16)] = a1
            return c2

        lax.fori_loop(0, _QCHUNK, q_body, 0)
        pltpu.sync_copy(out_v, out.at[b, pl.ds(g * _QCHUNK, _QCHUNK), h])

    issue(0, 0)

    def pair_body(o, carry):
        g0 = 2 * o
        issue(g0 + 1, 1)
        drain(0)
        compute(g0, 0)

        @pl.when(o < _NCHUNK // 2 - 1)
        def _():
            issue(g0 + 2, 0)

        drain(1)
        compute(g0 + 1, 1)
        return carry

    lax.fori_loop(0, _NCHUNK // 2, pair_body, 0)


@functools.lru_cache(maxsize=None)
def _sc_sample_fn():
    return pl.kernel(
        _sc_sample_body,
        out_type=jax.ShapeDtypeStruct((_BS, _LQ, _NH, _HD), jnp.float32),
        mesh=plsc.VectorSubcoreMesh(core_axis_name="c", subcore_axis_name="s"),
        scratch_types=[
            pltpu.VMEM((2, _NSUB, _GSUB), jnp.int32),
            pltpu.VMEM((2, _CT), jnp.float32),
            pltpu.VMEM((2, _CT, _HD), jnp.float32),
            pltpu.VMEM((_QCHUNK, _HD), jnp.float32),
            (pltpu.SemaphoreType.DMA, pltpu.SemaphoreType.DMA),
        ],
        compiler_params=pltpu.CompilerParams(use_tc_tiling_on_sc=False),
    )


def _sc_sample(table, idxs, wts):
    return _sc_sample_fn()(table, idxs, wts)


def _build_taps(so, aw, rp_l, rp_r):
    bs, lq = _BS, _LQ
    rpl = jnp.transpose(rp_l, (0, 1, 3, 2, 4)).reshape(bs, lq, _NL, 2)
    rpr = jnp.transpose(rp_r, (0, 1, 3, 2, 4)).reshape(bs, lq, _NL, 2)
    wdim = jnp.asarray([float(w) for _, w in _SHAPES], jnp.float32)
    hdim = jnp.asarray([float(h) for h, _ in _SHAPES], jnp.float32)
    norm = jnp.stack([wdim, hdim], -1)

    rpl_b = rpl[:, None, :, :, None, :]
    rpr_b = rpr[:, None, :, :, None, :]
    nrm = norm[None, None, None, :, None, :]
    locs_l = rpl_b + so[..., :2] / nrm
    locs_r = rpr_b + so[..., 2:] / nrm
    locs = jnp.concatenate([locs_l, locs_r], axis=-2)

    ix = locs[..., 0] * wdim[None, None, None, :, None] - 0.5
    iy = locs[..., 1] * hdim[None, None, None, :, None] - 0.5
    x0f = jnp.floor(ix)
    y0f = jnp.floor(iy)
    wx = ix - x0f
    wy = iy - y0f
    x0 = x0f.astype(jnp.int32)
    y0 = y0f.astype(jnp.int32)

    aw2 = jnp.concatenate([aw, aw], axis=-1)
    wi = jnp.asarray([w for _, w in _SHAPES], jnp.int32)[None, None, None, :, None]
    hi = jnp.asarray([h for h, _ in _SHAPES], jnp.int32)[None, None, None, :, None]
    offs = jnp.asarray(_OFFS, jnp.int32)[None, None, None, :, None]
    boff = (jnp.arange(bs, dtype=jnp.int32) * _LEN_V)[:, None, None, None, None]
    hoff = jnp.arange(_NH, dtype=jnp.int32)[None, :, None, None, None]

    idx_taps, wt_taps = [], []
    for dy in (0, 1):
        for dx in (0, 1):
            xt = x0 + dx
            yt = y0 + dy
            valid = ((xt >= 0) & (xt < wi) & (yt >= 0) & (yt < hi))
            wtap = ((wx if dx else (1.0 - wx)) * (wy if dy else (1.0 - wy))
                    * valid.astype(jnp.float32) * aw2)
            pix = (boff + offs
                   + jnp.clip(yt, 0, hi - 1) * wi
                   + jnp.clip(xt, 0, wi - 1))
            idx_taps.append(pix * _NH + hoff)
            wt_taps.append(wtap)

    idx = jnp.stack(idx_taps, axis=-1)
    wts = jnp.stack(wt_taps, axis=-1)
    idx = idx.reshape(_NW, _NCHUNK, _NSUB, _GSUB)
    wts = wts.reshape(_NW, lq * _TAPS)
    return idx, wts, locs_l, locs_r


def kernel(query, reference_points_l, reference_points_r, value,
           value_spatial_shapes, W_off, b_off, W_attn, b_attn, W_val, b_val,
           W_out, b_out):
    del value_spatial_shapes
    bs, lq, _ = query.shape

    q2d = query.reshape(bs * lq, _EMBED)
    so2d, aw2d = _tc_qproj(q2d, W_off, b_off, W_attn, b_attn)
    so = jnp.transpose(so2d.reshape(bs, lq, _NH, _NL * _NP * 4), (0, 2, 1, 3))
    so = so.reshape(bs, _NH, lq, _NL, _NP, 4)
    aw = jnp.transpose(aw2d.reshape(bs, lq, _NH, _NL * _NP), (0, 2, 1, 3))
    aw = aw.reshape(bs, _NH, lq, _NL, _NP)

    v2d = _tc_matmul_bias(value.reshape(bs * _LEN_V, 2 * _EMBED),
                          W_val, b_val, 480)
    table = v2d.reshape(bs * _LEN_V * _NH, _HD)

    idx, wts, locs_l, locs_r = _build_taps(so, aw, reference_points_l,
                                           reference_points_r)

    sampled = _sc_sample(table, idx, wts)
    out2d = sampled.reshape(bs * lq, _EMBED)
    out = _tc_matmul_bias(out2d, W_out, b_out, 240).reshape(bs, lq, _EMBED)

    tp = _NH * _NL * _NP
    sl_l = jnp.transpose(locs_l[:, :, ::_NK], (0, 2, 1, 3, 4, 5)).reshape(
        bs, _NQ, tp, 2)
    sl_r = jnp.transpose(locs_r[:, :, ::_NK], (0, 2, 1, 3, 4, 5)).reshape(
        bs, _NQ, tp, 2)
    aw_log = jnp.transpose(aw[:, :, ::_NK], (0, 2, 1, 3, 4)).reshape(
        bs, _NQ, tp)
    log_info = (lax.stop_gradient(sl_l), lax.stop_gradient(aw_log),
                lax.stop_gradient(sl_r), lax.stop_gradient(aw_log))
    return out, log_info
